# Initial kernel scaffold; baseline (speedup 1.0000x reference)
import jax
import jax.numpy as jnp
from jax.experimental import pallas as pl


def kernel(x, edge_index, W, b, gamma, beta):
    raise NotImplementedError("write your pallas kernel here")


# speedup vs baseline: 12.7155x; 12.7155x over previous
import functools

import jax
import jax.numpy as jnp
from jax import lax
from jax.experimental import pallas as pl
from jax.experimental.pallas import tpu as pltpu
from jax.experimental.pallas import tpu_sc as plsc

N = 10000
E = 160000
D = 256
HD = D // 2

CH = 128
NSUB = 16
CHM = -(-E // (NSUB * CH))
EM = NSUB * CH * CHM
CHD = -(-E // (2 * NSUB * CH))
ED = 2 * NSUB * CH * CHD
NPAD = 10240
DEG_SH = NPAD // NSUB
ROWS_T = 632
NPAD2 = NSUB * ROWS_T
TAIL = N - 15 * ROWS_T

_mesh = plsc.VectorSubcoreMesh(core_axis_name="c", subcore_axis_name="s")


@functools.partial(
    pl.kernel,
    out_type=[jax.ShapeDtypeStruct((NPAD,), jnp.float32),
              jax.ShapeDtypeStruct((NPAD,), jnp.float32)],
    mesh=_mesh,
    scratch_types=[pltpu.VMEM((CHD, CH), jnp.int32),
                   pltpu.VMEM((CH,), jnp.float32),
                   pltpu.VMEM_SHARED((NPAD,), jnp.float32)],
)
def _deg_kernel(dst_hbm, zeros_hbm, ones_hbm, d0_hbm, d1_hbm,
                idx_v, ones_v, deg_sh):
    c = lax.axis_index("c")
    s = lax.axis_index("s")
    wid = c * NSUB + s
    pltpu.sync_copy(dst_hbm.at[wid], idx_v)
    pltpu.sync_copy(ones_hbm, ones_v)
    pltpu.sync_copy(zeros_hbm.at[pl.ds(s * DEG_SH, DEG_SH)],
                    deg_sh.at[pl.ds(s * DEG_SH, DEG_SH)])
    plsc.subcore_barrier()

    @pl.loop(0, CHD)
    def _(j):
        pltpu.sync_copy(ones_v, deg_sh.at[idx_v.at[j]], add=True)

    plsc.subcore_barrier()

    @pl.when(c == 0)
    def _():
        pltpu.sync_copy(deg_sh.at[pl.ds(s * DEG_SH, DEG_SH)],
                        d0_hbm.at[pl.ds(s * DEG_SH, DEG_SH)])

    @pl.when(c == 1)
    def _():
        pltpu.sync_copy(deg_sh.at[pl.ds(s * DEG_SH, DEG_SH)],
                        d1_hbm.at[pl.ds(s * DEG_SH, DEG_SH)])


@functools.partial(
    pl.kernel,
    out_type=[jax.ShapeDtypeStruct((N, HD), jnp.float32),
              jax.ShapeDtypeStruct((N, HD), jnp.float32)],
    mesh=_mesh,
    scratch_types=[pltpu.VMEM((CHM, CH), jnp.int32),
                   pltpu.VMEM((CHM, CH), jnp.int32),
                   pltpu.VMEM((CH, HD), jnp.float32),
                   pltpu.VMEM_SHARED((NPAD2, HD), jnp.float32)],
)
def _agg_kernel(g0_hbm, g1_hbm, src_hbm, dst_hbm, a0_hbm, a1_hbm,
                src_v, dst_v, buf, agg_sh):
    c = lax.axis_index("c")
    s = lax.axis_index("s")
    pltpu.sync_copy(src_hbm.at[s], src_v)
    pltpu.sync_copy(dst_hbm.at[s], dst_v)

    def run(g_hbm, out_hbm):
        @pl.when(s < 15)
        def _():
            pltpu.sync_copy(g_hbm.at[pl.ds(s * ROWS_T, ROWS_T)],
                            agg_sh.at[pl.ds(s * ROWS_T, ROWS_T)])

        @pl.when(s == 15)
        def _():
            pltpu.sync_copy(g_hbm.at[pl.ds(15 * ROWS_T, TAIL)],
                            agg_sh.at[pl.ds(15 * ROWS_T, TAIL)])

        plsc.subcore_barrier()

        @pl.loop(0, CHM)
        def _(j):
            pltpu.sync_copy(g_hbm.at[src_v.at[j]], buf)
            pltpu.sync_copy(buf, agg_sh.at[dst_v.at[j]], add=True)

        plsc.subcore_barrier()

        @pl.when(s < 15)
        def _():
            pltpu.sync_copy(agg_sh.at[pl.ds(s * ROWS_T, ROWS_T)],
                            out_hbm.at[pl.ds(s * ROWS_T, ROWS_T)])

        @pl.when(s == 15)
        def _():
            pltpu.sync_copy(agg_sh.at[pl.ds(15 * ROWS_T, TAIL)],
                            out_hbm.at[pl.ds(15 * ROWS_T, TAIL)])

    @pl.when(c == 0)
    def _():
        run(g0_hbm, a0_hbm)

    @pl.when(c == 1)
    def _():
        run(g1_hbm, a1_hbm)


BM = 1000


def _mm_body(x_ref, w_ref, d0_ref, d1_ref, g0_ref, g1_ref, dinv_ref):
    deg = d0_ref[...] + d1_ref[...] + 1.0
    dinv = lax.rsqrt(deg)
    h = jnp.dot(x_ref[...], w_ref[...], preferred_element_type=jnp.float32)
    g = h * dinv
    g0_ref[...] = g[:, :HD]
    g1_ref[...] = g[:, HD:]
    dinv_ref[...] = dinv


_mm_call = pl.pallas_call(
    _mm_body,
    grid=(N // BM,),
    in_specs=[pl.BlockSpec((BM, D), lambda i: (i, 0)),
              pl.BlockSpec((D, D), lambda i: (0, 0)),
              pl.BlockSpec((BM, 1), lambda i: (i, 0)),
              pl.BlockSpec((BM, 1), lambda i: (i, 0))],
    out_specs=[pl.BlockSpec((BM, HD), lambda i: (i, 0)),
               pl.BlockSpec((BM, HD), lambda i: (i, 0)),
               pl.BlockSpec((BM, 1), lambda i: (i, 0))],
    out_shape=[jax.ShapeDtypeStruct((N, HD), jnp.float32),
               jax.ShapeDtypeStruct((N, HD), jnp.float32),
               jax.ShapeDtypeStruct((N, 1), jnp.float32)],
)


def _stat_body(a0_ref, a1_ref, dinv_ref, b_ref, sum_ref, sq_ref):
    i = pl.program_id(0)
    o = jnp.concatenate([a0_ref[...], a1_ref[...]], axis=1)
    o = o * dinv_ref[...] + b_ref[...]
    ssum = jnp.sum(o, axis=0, keepdims=True)
    ssq = jnp.sum(o * o, axis=0, keepdims=True)

    @pl.when(i == 0)
    def _():
        sum_ref[...] = ssum
        sq_ref[...] = ssq

    @pl.when(i > 0)
    def _():
        sum_ref[...] += ssum
        sq_ref[...] += ssq


_stat_call = pl.pallas_call(
    _stat_body,
    grid=(N // BM,),
    in_specs=[pl.BlockSpec((BM, HD), lambda i: (i, 0)),
              pl.BlockSpec((BM, HD), lambda i: (i, 0)),
              pl.BlockSpec((BM, 1), lambda i: (i, 0)),
              pl.BlockSpec((1, D), lambda i: (0, 0))],
    out_specs=[pl.BlockSpec((1, D), lambda i: (0, 0)),
               pl.BlockSpec((1, D), lambda i: (0, 0))],
    out_shape=[jax.ShapeDtypeStruct((1, D), jnp.float32),
               jax.ShapeDtypeStruct((1, D), jnp.float32)],
)


def _apply_body(a0_ref, a1_ref, dinv_ref, b_ref, gamma_ref, beta_ref,
                sum_ref, sq_ref, out_ref):
    mean = sum_ref[...] * (1.0 / N)
    var = sq_ref[...] * (1.0 / N) - mean * mean
    scale = gamma_ref[...] * lax.rsqrt(var + 1e-5)
    shift = beta_ref[...] - mean * scale
    o = jnp.concatenate([a0_ref[...], a1_ref[...]], axis=1)
    o = (o * dinv_ref[...] + b_ref[...]) * scale + shift
    out_ref[...] = jnp.maximum(o, 0.0)


_apply_call = pl.pallas_call(
    _apply_body,
    grid=(N // BM,),
    in_specs=[pl.BlockSpec((BM, HD), lambda i: (i, 0)),
              pl.BlockSpec((BM, HD), lambda i: (i, 0)),
              pl.BlockSpec((BM, 1), lambda i: (i, 0)),
              pl.BlockSpec((1, D), lambda i: (0, 0)),
              pl.BlockSpec((1, D), lambda i: (0, 0)),
              pl.BlockSpec((1, D), lambda i: (0, 0)),
              pl.BlockSpec((1, D), lambda i: (0, 0)),
              pl.BlockSpec((1, D), lambda i: (0, 0))],
    out_specs=pl.BlockSpec((BM, D), lambda i: (i, 0)),
    out_shape=jax.ShapeDtypeStruct((N, D), jnp.float32),
)


def kernel(x, edge_index, W, b, gamma, beta):
    src = edge_index[0]
    dst = edge_index[1]

    src_m = jnp.concatenate(
        [src, jnp.zeros((EM - E,), jnp.int32)]).reshape(NSUB, CHM, CH)
    dst_m = jnp.concatenate(
        [dst, jnp.full((EM - E,), N, jnp.int32)]).reshape(NSUB, CHM, CH)
    dst_d = jnp.concatenate(
        [dst, jnp.full((ED - E,), N, jnp.int32)]).reshape(2 * NSUB, CHD, CH)

    zeros_np = jnp.zeros((NPAD,), jnp.float32)
    ones_ch = jnp.ones((CH,), jnp.float32)

    d0, d1 = _deg_kernel(dst_d, zeros_np, ones_ch)
    d0 = d0[:N].reshape(N, 1)
    d1 = d1[:N].reshape(N, 1)

    g0, g1, dinv = _mm_call(x, W, d0, d1)
    a0, a1 = _agg_kernel(g0, g1, src_m, dst_m)

    b2 = b.reshape(1, D)
    gamma2 = gamma.reshape(1, D)
    beta2 = beta.reshape(1, D)
    ssum, ssq = _stat_call(a0, a1, dinv, b2)
    return _apply_call(a0, a1, dinv, b2, gamma2, beta2, ssum, ssq)
